# baseline (speedup 1.0000x reference)
import functools

import jax
import jax.numpy as jnp
from jax import lax
from jax.experimental import pallas as pl
from jax.experimental.pallas import tpu as pltpu
from jax.experimental.pallas import tpu_sc as plsc

_EOS = 2
_ROWS = 128
_VOCAB = 100000
_IDS_LEN = 8192
_GR = 8
_CHUNK_C = 12288
_IDS_CHUNK = 2048
_LANES = 16
_SC_COLS = (_VOCAB // 128) * 128

_CHUNKS = []
_c = 0
while _c + _CHUNK_C <= _SC_COLS:
    _CHUNKS.append((_c, _CHUNK_C))
    _c += _CHUNK_C
if _c < _SC_COLS:
    _CHUNKS.append((_c, _SC_COLS - _c))


def _sc_body(ids_hbm, scores_hbm, out_hbm, ids_v, buf_v, acc_v):
    wid = lax.axis_index("s") * 2 + lax.axis_index("c")
    half = wid % 2
    base_row = pl.multiple_of((wid // 2) * _GR, _GR)

    ones = jnp.ones((_LANES,), jnp.int32)
    zeros = jnp.zeros((_LANES,), jnp.int32)
    for r in range(_GR):
        acc_v[pl.ds(r * _LANES, _LANES)] = zeros
    for c0 in range(0, _IDS_LEN, _IDS_CHUNK):
        pltpu.sync_copy(
            ids_hbm.at[pl.ds(base_row, _GR), pl.ds(c0, _IDS_CHUNK)], ids_v)
        for r in range(_GR):
            def count_step(j, carry, r=r):
                v = ids_v[r, pl.ds(j * _LANES, _LANES)]
                sl = pl.ds(r * _LANES, _LANES)
                acc_v[sl] = acc_v[sl] + jnp.where(v == _EOS, ones, zeros)
                return carry
            lax.fori_loop(0, _IDS_CHUNK // _LANES, count_step, 0)
    counts = []
    for r in range(_GR):
        vec = acc_v[pl.ds(r * _LANES, _LANES)]
        c = vec[0]
        for j in range(1, _LANES):
            c = c + vec[j]
        counts.append(c)
    counts_init = counts
    done = [(c - ci) >= 2 for c, ci in zip(counts, counts_init)]

    for k, (c0, w) in enumerate(_CHUNKS):
        @pl.when(half == (k % 2))
        def _relay(c0=c0, w=w):
            src = scores_hbm.at[pl.ds(base_row, _GR), pl.ds(c0, w)]
            dst = out_hbm.at[pl.ds(base_row, _GR), pl.ds(c0, w)]
            buf = buf_v.at[:, pl.ds(0, w)]
            pltpu.sync_copy(src, buf)
            for r in range(_GR):
                @pl.when(done[r])
                def _mask(r=r, w=w, c0=c0):
                    neg_inf = jnp.full((_LANES,), -jnp.inf, jnp.float32)

                    def mask_step(j, carry):
                        buf_v[r, pl.ds(j * _LANES, _LANES)] = neg_inf
                        return carry
                    lax.fori_loop(0, w // _LANES, mask_step, 0)
                    if c0 == 0:
                        head = buf_v[r, pl.ds(0, _LANES)]
                        eos_pos = lax.iota(jnp.int32, _LANES) == _EOS
                        buf_v[r, pl.ds(0, _LANES)] = jnp.where(
                            eos_pos, jnp.zeros((_LANES,), jnp.float32), head)
            pltpu.sync_copy(buf, dst)


def _tc_tail_body(ids_ref, tail_ref, _alias_ref, out_ref):
    counts = jnp.sum((ids_ref[...] == _EOS).astype(jnp.int32), axis=1,
                     keepdims=True)
    count_init = counts
    done = (counts - count_init) >= 2
    out_ref[...] = jnp.where(done, -jnp.inf, tail_ref[...])


def kernel(input_ids, scores):
    mesh = plsc.VectorSubcoreMesh(core_axis_name="c", subcore_axis_name="s")
    sc_run = functools.partial(
        pl.kernel,
        out_type=jax.ShapeDtypeStruct(scores.shape, scores.dtype),
        mesh=mesh,
        scratch_types=[
            pltpu.VMEM((_GR, _IDS_CHUNK), jnp.int32),
            pltpu.VMEM((_GR, _CHUNK_C), jnp.float32),
            pltpu.VMEM((_GR * _LANES,), jnp.int32),
        ],
    )(_sc_body)
    bulk = sc_run(input_ids, scores)

    tail_block = _SC_COLS // 128
    return pl.pallas_call(
        _tc_tail_body,
        grid=(1,),
        in_specs=[
            pl.BlockSpec(input_ids.shape, lambda i: (0, 0)),
            pl.BlockSpec((_ROWS, 128), lambda i: (0, tail_block)),
            pl.BlockSpec(memory_space=pl.ANY),
        ],
        out_specs=pl.BlockSpec((_ROWS, 128), lambda i: (0, tail_block)),
        out_shape=jax.ShapeDtypeStruct(scores.shape, scores.dtype),
        input_output_aliases={2: 0},
    )(input_ids, scores, bulk)


# speedup vs baseline: 1.8780x; 1.8780x over previous
import jax
import jax.numpy as jnp
from jax.experimental import pallas as pl
from jax.experimental.pallas import tpu as pltpu

_EOS = 2
_CR = 8


def _eos_kernel(ids_ref, scores_alias, out_hbm, done_ref, buf_ref, sem):
    rows = ids_ref.shape[0]
    counts = jnp.sum((ids_ref[...] == _EOS).astype(jnp.int32), axis=1,
                     keepdims=True)
    count_init = counts
    done = (counts - count_init) >= 2
    done_ref[...] = done.astype(jnp.float32)
    n_done = jnp.sum(done.astype(jnp.int32))

    @pl.when(n_done != 0)
    def _masked():
        for c in range(rows // _CR):
            cp_in = pltpu.make_async_copy(
                out_hbm.at[pl.ds(c * _CR, _CR), :], buf_ref, sem)
            cp_in.start()
            cp_in.wait()
            done_c = done_ref[pl.ds(c * _CR, _CR), :] > 0.0
            block = buf_ref[...]
            masked = jnp.where(done_c, -jnp.inf, block)
            buf_ref[...] = masked
            buf_ref[:, _EOS:_EOS + 1] = jnp.where(
                done_c, 0.0, block[:, _EOS:_EOS + 1])
            cp_out = pltpu.make_async_copy(
                buf_ref, out_hbm.at[pl.ds(c * _CR, _CR), :], sem)
            cp_out.start()
            cp_out.wait()


def kernel(input_ids, scores):
    batch, vocab = scores.shape
    return pl.pallas_call(
        _eos_kernel,
        in_specs=[
            pl.BlockSpec(input_ids.shape, lambda: (0, 0)),
            pl.BlockSpec(memory_space=pl.ANY),
        ],
        out_specs=pl.BlockSpec(memory_space=pl.ANY),
        out_shape=jax.ShapeDtypeStruct(scores.shape, scores.dtype),
        input_output_aliases={1: 0},
        scratch_shapes=[
            pltpu.VMEM((batch, 1), jnp.float32),
            pltpu.VMEM((_CR, vocab), jnp.float32),
            pltpu.SemaphoreType.DMA,
        ],
    )(input_ids, scores)


# speedup vs baseline: 1.8822x; 1.0023x over previous
import jax
import jax.numpy as jnp
from jax.experimental import pallas as pl
from jax.experimental.pallas import tpu as pltpu

_EOS = 2
_CR = 8


def _eos_kernel(ids_ref, scores_alias, out_hbm, done_ref, buf_ref, sem):
    rows = ids_ref.shape[0]
    counts = jnp.sum((ids_ref[...] == _EOS).astype(jnp.int32), axis=1,
                     keepdims=True)
    count_init = counts
    done = (counts - count_init) >= 2
    done_ref[...] = done.astype(jnp.float32)
    n_done = jnp.sum(done.astype(jnp.int32))

    @pl.when(n_done != 0)
    def _masked():
        for c in range(rows // _CR):
            cp_in = pltpu.make_async_copy(
                out_hbm.at[pl.ds(c * _CR, _CR), :], buf_ref, sem)
            cp_in.start()
            cp_in.wait()
            done_c = done_ref[pl.ds(c * _CR, _CR), :] > 0.0
            block = buf_ref[...]
            masked = jnp.where(done_c, -jnp.inf, block)
            buf_ref[...] = masked
            buf_ref[:, _EOS:_EOS + 1] = jnp.where(
                done_c, 0.0, block[:, _EOS:_EOS + 1])
            cp_out = pltpu.make_async_copy(
                buf_ref, out_hbm.at[pl.ds(c * _CR, _CR), :], sem)
            cp_out.start()
            cp_out.wait()


def kernel(input_ids, scores):
    batch, vocab = scores.shape
    scores = jnp.copy(scores)
    return pl.pallas_call(
        _eos_kernel,
        in_specs=[
            pl.BlockSpec(input_ids.shape, lambda: (0, 0)),
            pl.BlockSpec(memory_space=pl.ANY),
        ],
        out_specs=pl.BlockSpec(memory_space=pl.ANY),
        out_shape=jax.ShapeDtypeStruct(scores.shape, scores.dtype),
        input_output_aliases={1: 0},
        scratch_shapes=[
            pltpu.VMEM((batch, 1), jnp.float32),
            pltpu.VMEM((_CR, vocab), jnp.float32),
            pltpu.SemaphoreType.DMA,
        ],
    )(input_ids, scores)


# speedup vs baseline: 1.8837x; 1.0008x over previous
import jax
import jax.numpy as jnp
from jax.experimental import pallas as pl
from jax.experimental.pallas import tpu as pltpu

_EOS = 2
_CR = 8


def _eos_kernel(ids_ref, scores_alias, out_hbm, done_ref, buf_ref, sem):
    rows = ids_ref.shape[0]
    counts = jnp.sum((ids_ref[...] == _EOS).astype(jnp.int32), axis=1,
                     keepdims=True)
    count_init = counts
    done = (counts - count_init) >= 2
    done_ref[...] = done.astype(jnp.float32)
    n_done = jnp.sum(done.astype(jnp.int32))

    @pl.when(n_done != 0)
    def _masked():
        for c in range(rows // _CR):
            cp_in = pltpu.make_async_copy(
                out_hbm.at[pl.ds(c * _CR, _CR), :], buf_ref, sem)
            cp_in.start()
            cp_in.wait()
            done_c = done_ref[pl.ds(c * _CR, _CR), :] > 0.0
            block = buf_ref[...]
            masked = jnp.where(done_c, -jnp.inf, block)
            buf_ref[...] = masked
            buf_ref[:, _EOS:_EOS + 1] = jnp.where(
                done_c, 0.0, block[:, _EOS:_EOS + 1])
            cp_out = pltpu.make_async_copy(
                buf_ref, out_hbm.at[pl.ds(c * _CR, _CR), :], sem)
            cp_out.start()
            cp_out.wait()


def kernel(input_ids, scores):
    batch, vocab = scores.shape
    scores = scores + jnp.float32(0.0)
    return pl.pallas_call(
        _eos_kernel,
        in_specs=[
            pl.BlockSpec(input_ids.shape, lambda: (0, 0)),
            pl.BlockSpec(memory_space=pl.ANY),
        ],
        out_specs=pl.BlockSpec(memory_space=pl.ANY),
        out_shape=jax.ShapeDtypeStruct(scores.shape, scores.dtype),
        input_output_aliases={1: 0},
        scratch_shapes=[
            pltpu.VMEM((batch, 1), jnp.float32),
            pltpu.VMEM((_CR, vocab), jnp.float32),
            pltpu.SemaphoreType.DMA,
        ],
    )(input_ids, scores)
